# baseline (speedup 1.0000x reference)
import functools

import jax
import jax.numpy as jnp
from jax import lax
from jax.experimental import pallas as pl
from jax.experimental.pallas import tpu as pltpu
from jax.experimental.pallas import tpu_sc as plsc

try:
    _info = plsc.get_sparse_core_info()
    _NS = _info.num_subcores
except Exception:
    _NS = 16
_NW = _NS

_CHUNK = 128
_SLOTS = 7


def _build_embed(B, V, D, b_per_w, n_chunks):
    mesh = plsc.VectorSubcoreMesh(
        core_axis_name="c", subcore_axis_name="s", num_cores=1
    )

    @functools.partial(
        pl.kernel,
        mesh=mesh,
        out_type=jax.ShapeDtypeStruct((B, D), jnp.float32),
        scratch_types=[
            pltpu.VMEM((n_chunks, _CHUNK), jnp.int32),
            pltpu.VMEM((_SLOTS * _CHUNK, D), jnp.float32),
            pltpu.SemaphoreType.DMA((n_chunks,)),
            pltpu.SemaphoreType.DMA,
            pltpu.SemaphoreType.DMA,
            pltpu.SemaphoreType.DMA,
        ],
    )
    def _embed(table_hbm, idx_hbm, out_hbm, idx_v, rows_v, gsem, osem, w0sem,
               isem):
        wid = lax.axis_index("s")
        base = wid * b_per_w

        def gather(j, slot):
            return pltpu.async_copy(
                table_hbm.at[idx_v.at[j]],
                rows_v.at[pl.ds(slot * _CHUNK, _CHUNK)],
                gsem.at[j],
            )

        i0 = pltpu.async_copy(idx_hbm.at[wid].at[pl.ds(0, 1)],
                              idx_v.at[pl.ds(0, 1)], isem)
        i1 = pltpu.async_copy(idx_hbm.at[wid].at[pl.ds(1, n_chunks - 1)],
                              idx_v.at[pl.ds(1, n_chunks - 1)], isem)
        i0.wait()
        gathers = [gather(0, 0)]
        i1.wait()
        gathers += [gather(j, j) for j in range(1, _SLOTS)]
        gathers[0].wait()
        w0 = pltpu.async_copy(
            rows_v.at[pl.ds(0, _CHUNK)], out_hbm.at[pl.ds(base, _CHUNK)],
            w0sem,
        )
        for j in range(1, _SLOTS):
            gathers[j].wait()
        wmid = pltpu.async_copy(
            rows_v.at[pl.ds(_CHUNK, (_SLOTS - 1) * _CHUNK)],
            out_hbm.at[pl.ds(base + _CHUNK, (_SLOTS - 1) * _CHUNK)],
            osem,
        )
        w0.wait()
        glast = gather(_SLOTS, 0)
        glast.wait()
        wlast = pltpu.async_copy(
            rows_v.at[pl.ds(0, _CHUNK)],
            out_hbm.at[pl.ds(base + _SLOTS * _CHUNK, _CHUNK)],
            w0sem,
        )
        wmid.wait()
        wlast.wait()

    return _embed


@jax.jit
def kernel(labels, embedding):
    (B,) = labels.shape
    V, D = embedding.shape
    b_per_w = B // _NW
    n_chunks = b_per_w // _CHUNK
    idx = labels.astype(jnp.int32).reshape(_NW, n_chunks, _CHUNK)
    return _build_embed(B, V, D, b_per_w, n_chunks)(embedding, idx)


# speedup vs baseline: 1.1020x; 1.1020x over previous
import functools

import jax
import jax.numpy as jnp
from jax import lax
from jax.experimental import pallas as pl
from jax.experimental.pallas import tpu as pltpu
from jax.experimental.pallas import tpu_sc as plsc

try:
    _info = plsc.get_sparse_core_info()
    _NC, _NS = _info.num_cores, _info.num_subcores
except Exception:
    _NC, _NS = 2, 16
_NW = _NC * _NS

_CHUNK = 128


def _build_embed(B, V, D, b_per_w, n_chunks):
    mesh = plsc.VectorSubcoreMesh(core_axis_name="c", subcore_axis_name="s")

    @functools.partial(
        pl.kernel,
        mesh=mesh,
        out_type=jax.ShapeDtypeStruct((B, D), jnp.float32),
        scratch_types=[
            pltpu.VMEM((n_chunks, _CHUNK), jnp.int32),
            pltpu.VMEM((b_per_w, D), jnp.float32),
            pltpu.SemaphoreType.DMA,
            pltpu.SemaphoreType.DMA,
            pltpu.SemaphoreType.DMA,
        ],
    )
    def _embed(table_hbm, idx_hbm, out_hbm, idx_v, rows_v, gsem, osem, isem):
        wid = lax.axis_index("s") * _NC + lax.axis_index("c")

        def gather(j):
            return pltpu.async_copy(
                table_hbm.at[idx_v.at[j]],
                rows_v.at[pl.ds(j * _CHUNK, _CHUNK)],
                gsem,
            )

        i0 = pltpu.async_copy(idx_hbm.at[wid].at[pl.ds(0, 1)],
                              idx_v.at[pl.ds(0, 1)], isem)
        i1 = pltpu.async_copy(idx_hbm.at[wid].at[pl.ds(1, n_chunks - 1)],
                              idx_v.at[pl.ds(1, n_chunks - 1)], isem)
        i0.wait()
        gathers = [gather(0)]
        i1.wait()
        gathers += [gather(j) for j in range(1, n_chunks)]
        for g in gathers:
            g.wait()
        pltpu.async_copy(
            rows_v, out_hbm.at[pl.ds(wid * b_per_w, b_per_w)], osem
        ).wait()

    return _embed


@jax.jit
def kernel(labels, embedding):
    (B,) = labels.shape
    V, D = embedding.shape
    b_per_w = B // _NW
    n_chunks = b_per_w // _CHUNK
    idx = labels.astype(jnp.int32).reshape(_NW, n_chunks, _CHUNK)
    return _build_embed(B, V, D, b_per_w, n_chunks)(embedding, idx)
